# baseline (speedup 1.0000x reference)
import functools

import jax
import jax.numpy as jnp
from jax import lax
from jax.experimental import pallas as pl
from jax.experimental.pallas import tpu as pltpu
from jax.experimental.pallas import tpu_sc as plsc

N = 10000
E = 320000
D = 128

NC = 2
NS = 16
NW = NC * NS
CH = 64
EPW = 10240
NCHUNK = EPW // CH
SEG = 32
NSEG = NCHUNK // SEG
PAIRS = SEG // 2
N_PAD = 10112
RPT = N_PAD // NS

_BLK = 1000


def _tc1_body(x_ref, wla_t, w1_t, wlb_t, w2_t, wra_t, wrb_t, bp_ref,
              ya_ref, yb_ref, base_ref):
    f32 = jnp.float32
    xb = x_ref[...]
    ma = jnp.dot(wla_t[...], w1_t[...], preferred_element_type=f32)
    mb = jnp.dot(wlb_t[...], w2_t[...], preferred_element_type=f32)
    mc = (jnp.dot(wra_t[...], w1_t[...], preferred_element_type=f32)
          + jnp.dot(wrb_t[...], w2_t[...], preferred_element_type=f32))
    ya_ref[...] = jnp.dot(xb, ma, preferred_element_type=f32)
    yb_ref[...] = jnp.dot(xb, mb, preferred_element_type=f32)
    base_ref[...] = jnp.dot(xb, mc, preferred_element_type=f32) + bp_ref[...]


def _tc1(x, wla_t, w1_t, wlb_t, w2_t, wra_t, wrb_t, bprime):
    full = pl.BlockSpec((D, D), lambda i: (0, 0))
    return pl.pallas_call(
        _tc1_body,
        grid=(N // _BLK,),
        in_specs=[
            pl.BlockSpec((_BLK, D), lambda i: (i, 0)),
            full, full, full, full, full, full,
            pl.BlockSpec((1, D), lambda i: (0, 0)),
        ],
        out_specs=[
            pl.BlockSpec((_BLK, D), lambda i: (i, 0)),
            pl.BlockSpec((_BLK, D), lambda i: (i, 0)),
            pl.BlockSpec((_BLK, D), lambda i: (i, 0)),
        ],
        out_shape=[
            jax.ShapeDtypeStruct((N, D), jnp.float32),
            jax.ShapeDtypeStruct((N, D), jnp.float32),
            jax.ShapeDtypeStruct((N, D), jnp.float32),
        ],
    )(x, wla_t, w1_t, wlb_t, w2_t, wra_t, wrb_t, bprime)


_MESH = plsc.VectorSubcoreMesh(core_axis_name="c", subcore_axis_name="s")


@functools.partial(
    pl.kernel,
    out_type=jax.ShapeDtypeStruct((NC, N_PAD, D), jnp.float32),
    mesh=_MESH,
    scratch_types=[
        pltpu.VMEM_SHARED((N_PAD, D), jnp.float32),
        pltpu.VMEM((SEG, CH), jnp.int32),
        pltpu.VMEM((SEG, CH), jnp.int32),
        pltpu.VMEM((2, CH), jnp.int32),
        pltpu.VMEM((2, CH), jnp.int32),
        pltpu.VMEM((CH, D), jnp.float32),
        pltpu.VMEM((CH, D), jnp.float32),
        pltpu.VMEM((CH, D), jnp.float32),
        pltpu.VMEM((CH, D), jnp.float32),
        pltpu.SemaphoreType.DMA,
        pltpu.SemaphoreType.DMA,
        pltpu.SemaphoreType.DMA,
        pltpu.SemaphoreType.DMA,
        pltpu.SemaphoreType.DMA,
        pltpu.SemaphoreType.DMA,
        pltpu.SemaphoreType.DMA,
        pltpu.SemaphoreType.DMA,
    ],
)
def _sc_scatter(ya_hbm, yb_hbm, src_hbm, dst_hbm, zeros_hbm, out_hbm,
                acc, src_v, dst_v, mdst_v, msrc_v,
                rows_a0, rows_b0, rows_a1, rows_b1,
                ga0, gb0, ga1, gb1, sa0, sb0, sa1, sb1):
    c = lax.axis_index("c")
    s = lax.axis_index("s")
    wid = s * NC + c

    pltpu.sync_copy(zeros_hbm.at[pl.ds(s * RPT, RPT)],
                    acc.at[pl.ds(s * RPT, RPT)])
    plsc.subcore_barrier()

    dummy = N + s

    def mask_chunk(jj, r):
        for k in range(CH // 16):
            sl = pl.ds(k * 16, 16)
            sv = src_v[jj, sl]
            dv = dst_v[jj, sl]
            m = sv == dv
            mdst_v[r, sl] = jnp.where(m, dummy, dv)
            msrc_v[r, sl] = jnp.where(m, dummy, sv)

    dummy_vec = jnp.zeros((16,), jnp.int32) + dummy
    for r in range(2):
        for k in range(CH // 16):
            sl = pl.ds(k * 16, 16)
            mdst_v[r, sl] = dummy_vec
            msrc_v[r, sl] = dummy_vec
    pltpu.async_copy(rows_a0, acc.at[mdst_v.at[0]], sa0, add=True)
    pltpu.async_copy(rows_b0, acc.at[msrc_v.at[0]], sb0, add=True)
    pltpu.async_copy(rows_a1, acc.at[mdst_v.at[1]], sa1, add=True)
    pltpu.async_copy(rows_b1, acc.at[msrc_v.at[1]], sb1, add=True)

    def scat_wait(rows, ring, sem):
        pltpu.make_async_copy(rows, acc.at[ring], sem).wait()

    def pair(t, carry):
        j0 = 2 * t
        j1 = j0 + 1
        scat_wait(rows_a0, mdst_v.at[0], sa0)
        scat_wait(rows_b0, msrc_v.at[0], sb0)
        da0 = pltpu.async_copy(ya_hbm.at[src_v.at[j0]], rows_a0, ga0)
        db0 = pltpu.async_copy(yb_hbm.at[dst_v.at[j0]], rows_b0, gb0)
        mask_chunk(j0, 0)
        da0.wait()
        pltpu.async_copy(rows_a0, acc.at[mdst_v.at[0]], sa0, add=True)
        db0.wait()
        pltpu.async_copy(rows_b0, acc.at[msrc_v.at[0]], sb0, add=True)
        scat_wait(rows_a1, mdst_v.at[1], sa1)
        scat_wait(rows_b1, msrc_v.at[1], sb1)
        da1 = pltpu.async_copy(ya_hbm.at[src_v.at[j1]], rows_a1, ga1)
        db1 = pltpu.async_copy(yb_hbm.at[dst_v.at[j1]], rows_b1, gb1)
        mask_chunk(j1, 1)
        da1.wait()
        pltpu.async_copy(rows_a1, acc.at[mdst_v.at[1]], sa1, add=True)
        db1.wait()
        pltpu.async_copy(rows_b1, acc.at[msrc_v.at[1]], sb1, add=True)
        return carry

    def segment(g, carry):
        pltpu.sync_copy(src_hbm.at[wid, pl.ds(g * SEG, SEG)], src_v)
        pltpu.sync_copy(dst_hbm.at[wid, pl.ds(g * SEG, SEG)], dst_v)
        return lax.fori_loop(0, PAIRS, pair, carry)

    lax.fori_loop(0, NSEG, segment, 0)
    scat_wait(rows_a0, mdst_v.at[0], sa0)
    scat_wait(rows_b0, msrc_v.at[0], sb0)
    scat_wait(rows_a1, mdst_v.at[1], sa1)
    scat_wait(rows_b1, msrc_v.at[1], sb1)
    plsc.subcore_barrier()
    pltpu.sync_copy(acc.at[pl.ds(s * RPT, RPT)],
                    out_hbm.at[c, pl.ds(s * RPT, RPT)])


def _tc2_body(base_ref, a0_ref, a1_ref, out_ref):
    out_ref[...] = base_ref[...] + a0_ref[...] + a1_ref[...]


def _tc2(base, a0, a1):
    spec = pl.BlockSpec((_BLK, D), lambda i: (i, 0))
    return pl.pallas_call(
        _tc2_body,
        grid=(N // _BLK,),
        in_specs=[spec, spec, spec],
        out_specs=spec,
        out_shape=jax.ShapeDtypeStruct((N, D), jnp.float32),
    )(base, a0, a1)


@jax.jit
def kernel(x, edge_index, Wl_a, bl_a, Wr_a, Wl_b, bl_b, Wr_b, W, b):
    w1 = W[:, :D]
    w2 = W[:, D:]
    bprime = (b + w1 @ bl_a + w2 @ bl_b)[None, :]

    ya, yb, base = _tc1(x, Wl_a.T, w1.T, Wl_b.T, w2.T, Wr_a.T, Wr_b.T, bprime)

    pad = jnp.broadcast_to(jnp.arange(EPW - E // NW, dtype=jnp.int32) * 97 % N,
                           (NW, EPW - E // NW))
    src = jnp.concatenate([edge_index[0].reshape(NW, E // NW), pad], axis=1)
    dst = jnp.concatenate([edge_index[1].reshape(NW, E // NW), pad], axis=1)
    src3 = src.reshape(NW, NCHUNK, CH)
    dst3 = dst.reshape(NW, NCHUNK, CH)
    zeros = jnp.zeros((N_PAD, D), jnp.float32)

    accs = _sc_scatter(ya, yb, src3, dst3, zeros)

    return _tc2(base, accs[0, :N], accs[1, :N])


# speedup vs baseline: 1.0659x; 1.0659x over previous
import functools

import jax
import jax.numpy as jnp
from jax import lax
from jax.experimental import pallas as pl
from jax.experimental.pallas import tpu as pltpu
from jax.experimental.pallas import tpu_sc as plsc

N = 10000
E = 320000
D = 128

NC = 2
NS = 16
NW = NC * NS
CH = 32
EPW = 10240
NCHUNK = EPW // CH
SEG = 40
NSEG = NCHUNK // SEG
QUAD = 4
BODIES = SEG // QUAD
N_PAD = 10112
RPT = N_PAD // NS

_BLK = 1000


def _tc1_body(x_ref, wla_t, w1_t, wlb_t, w2_t, wra_t, wrb_t, bp_ref,
              ya_ref, yb_ref, base_ref):
    f32 = jnp.float32
    xb = x_ref[...]
    ma = jnp.dot(wla_t[...], w1_t[...], preferred_element_type=f32)
    mb = jnp.dot(wlb_t[...], w2_t[...], preferred_element_type=f32)
    mc = (jnp.dot(wra_t[...], w1_t[...], preferred_element_type=f32)
          + jnp.dot(wrb_t[...], w2_t[...], preferred_element_type=f32))
    ya_ref[...] = jnp.dot(xb, ma, preferred_element_type=f32)
    yb_ref[...] = jnp.dot(xb, mb, preferred_element_type=f32)
    base_ref[...] = jnp.dot(xb, mc, preferred_element_type=f32) + bp_ref[...]


def _tc1(x, wla_t, w1_t, wlb_t, w2_t, wra_t, wrb_t, bprime):
    full = pl.BlockSpec((D, D), lambda i: (0, 0))
    return pl.pallas_call(
        _tc1_body,
        grid=(N // _BLK,),
        in_specs=[
            pl.BlockSpec((_BLK, D), lambda i: (i, 0)),
            full, full, full, full, full, full,
            pl.BlockSpec((1, D), lambda i: (0, 0)),
        ],
        out_specs=[
            pl.BlockSpec((_BLK, D), lambda i: (i, 0)),
            pl.BlockSpec((_BLK, D), lambda i: (i, 0)),
            pl.BlockSpec((_BLK, D), lambda i: (i, 0)),
        ],
        out_shape=[
            jax.ShapeDtypeStruct((N, D), jnp.float32),
            jax.ShapeDtypeStruct((N, D), jnp.float32),
            jax.ShapeDtypeStruct((N, D), jnp.float32),
        ],
    )(x, wla_t, w1_t, wlb_t, w2_t, wra_t, wrb_t, bprime)


_MESH = plsc.VectorSubcoreMesh(core_axis_name="c", subcore_axis_name="s")


@functools.partial(
    pl.kernel,
    out_type=jax.ShapeDtypeStruct((NC, N_PAD, D), jnp.float32),
    mesh=_MESH,
    scratch_types=(
        [pltpu.VMEM_SHARED((N_PAD, D), jnp.float32)]
        + [pltpu.VMEM((SEG, CH), jnp.int32)] * 2
        + [pltpu.VMEM((QUAD, CH), jnp.int32)] * 2
        + [pltpu.VMEM((CH, D), jnp.float32)] * (2 * QUAD)
        + [pltpu.SemaphoreType.DMA] * (4 * QUAD)
    ),
)
def _sc_scatter(ya_hbm, yb_hbm, src_hbm, dst_hbm, zeros_hbm, out_hbm,
                acc, src_v, dst_v, mdst_v, msrc_v, *bufs):
    ra = bufs[0:QUAD]
    rb = bufs[QUAD:2 * QUAD]
    ga = bufs[2 * QUAD:3 * QUAD]
    gb = bufs[3 * QUAD:4 * QUAD]
    sa = bufs[4 * QUAD:5 * QUAD]
    sb = bufs[5 * QUAD:6 * QUAD]

    c = lax.axis_index("c")
    s = lax.axis_index("s")
    wid = s * NC + c

    pltpu.sync_copy(zeros_hbm.at[pl.ds(s * RPT, RPT)],
                    acc.at[pl.ds(s * RPT, RPT)])
    plsc.subcore_barrier()

    dummy = N + s

    def mask_chunk(jj, q):
        for k in range(CH // 16):
            sl = pl.ds(k * 16, 16)
            sv = src_v[jj, sl]
            dv = dst_v[jj, sl]
            m = sv == dv
            mdst_v[q, sl] = jnp.where(m, dummy, dv)
            msrc_v[q, sl] = jnp.where(m, dummy, sv)

    dummy_vec = jnp.zeros((16,), jnp.int32) + dummy
    for q in range(QUAD):
        for k in range(CH // 16):
            sl = pl.ds(k * 16, 16)
            mdst_v[q, sl] = dummy_vec
            msrc_v[q, sl] = dummy_vec
    for q in range(QUAD):
        pltpu.async_copy(ra[q], acc.at[mdst_v.at[q]], sa[q], add=True)
        pltpu.async_copy(rb[q], acc.at[msrc_v.at[q]], sb[q], add=True)

    def scat_wait(rows, ring, sem):
        pltpu.make_async_copy(rows, acc.at[ring], sem).wait()

    def quad(t, carry):
        j = QUAD * t
        da = []
        db = []
        for q in range(QUAD):
            scat_wait(ra[q], mdst_v.at[q], sa[q])
            da.append(pltpu.async_copy(ya_hbm.at[src_v.at[j + q]], ra[q], ga[q]))
            scat_wait(rb[q], msrc_v.at[q], sb[q])
            db.append(pltpu.async_copy(yb_hbm.at[dst_v.at[j + q]], rb[q], gb[q]))
            mask_chunk(j + q, q)
        for q in range(QUAD):
            da[q].wait()
            pltpu.async_copy(ra[q], acc.at[mdst_v.at[q]], sa[q], add=True)
            db[q].wait()
            pltpu.async_copy(rb[q], acc.at[msrc_v.at[q]], sb[q], add=True)
        return carry

    def segment(g, carry):
        pltpu.sync_copy(src_hbm.at[wid, pl.ds(g * SEG, SEG)], src_v)
        pltpu.sync_copy(dst_hbm.at[wid, pl.ds(g * SEG, SEG)], dst_v)
        return lax.fori_loop(0, BODIES, quad, carry)

    lax.fori_loop(0, NSEG, segment, 0)
    for q in range(QUAD):
        scat_wait(ra[q], mdst_v.at[q], sa[q])
        scat_wait(rb[q], msrc_v.at[q], sb[q])
    plsc.subcore_barrier()
    pltpu.sync_copy(acc.at[pl.ds(s * RPT, RPT)],
                    out_hbm.at[c, pl.ds(s * RPT, RPT)])


def _tc2_body(base_ref, a0_ref, a1_ref, out_ref):
    out_ref[...] = base_ref[...] + a0_ref[...] + a1_ref[...]


def _tc2(base, a0, a1):
    spec = pl.BlockSpec((_BLK, D), lambda i: (i, 0))
    return pl.pallas_call(
        _tc2_body,
        grid=(N // _BLK,),
        in_specs=[spec, spec, spec],
        out_specs=spec,
        out_shape=jax.ShapeDtypeStruct((N, D), jnp.float32),
    )(base, a0, a1)


@jax.jit
def kernel(x, edge_index, Wl_a, bl_a, Wr_a, Wl_b, bl_b, Wr_b, W, b):
    w1 = W[:, :D]
    w2 = W[:, D:]
    bprime = (b + w1 @ bl_a + w2 @ bl_b)[None, :]

    ya, yb, base = _tc1(x, Wl_a.T, w1.T, Wl_b.T, w2.T, Wr_a.T, Wr_b.T, bprime)

    pad = jnp.broadcast_to(jnp.arange(EPW - E // NW, dtype=jnp.int32) * 97 % N,
                           (NW, EPW - E // NW))
    src = jnp.concatenate([edge_index[0].reshape(NW, E // NW), pad], axis=1)
    dst = jnp.concatenate([edge_index[1].reshape(NW, E // NW), pad], axis=1)
    src3 = src.reshape(NW, NCHUNK, CH)
    dst3 = dst.reshape(NW, NCHUNK, CH)
    zeros = jnp.zeros((N_PAD, D), jnp.float32)

    accs = _sc_scatter(ya, yb, src3, dst3, zeros)

    return _tc2(base, accs[0, :N], accs[1, :N])


# speedup vs baseline: 1.0940x; 1.0263x over previous
import functools

import jax
import jax.numpy as jnp
from jax import lax
from jax.experimental import pallas as pl
from jax.experimental.pallas import tpu as pltpu
from jax.experimental.pallas import tpu_sc as plsc

N = 10000
E = 320000
D = 128

NC = 2
NS = 16
NW = NC * NS
CH = 32
EPW = 10240
NCHUNK = EPW // CH
SEG = 40
NSEG = NCHUNK // SEG
QUAD = 4
BODIES = SEG // QUAD
N_PAD = 10112
RPT = N_PAD // NS

_BLK = 1000


def _tc1_body(x_ref, wla_t, w1_t, wlb_t, w2_t, wra_t, wrb_t, bp_ref,
              ya_ref, yb_ref, base_ref):
    f32 = jnp.float32
    xb = x_ref[...]
    ma = jnp.dot(wla_t[...], w1_t[...], preferred_element_type=f32)
    mb = jnp.dot(wlb_t[...], w2_t[...], preferred_element_type=f32)
    mc = (jnp.dot(wra_t[...], w1_t[...], preferred_element_type=f32)
          + jnp.dot(wrb_t[...], w2_t[...], preferred_element_type=f32))
    ya_ref[...] = jnp.dot(xb, ma, preferred_element_type=f32)
    yb_ref[...] = jnp.dot(xb, mb, preferred_element_type=f32)
    base_ref[...] = jnp.dot(xb, mc, preferred_element_type=f32) + bp_ref[...]


def _tc1(x, wla_t, w1_t, wlb_t, w2_t, wra_t, wrb_t, bprime):
    full = pl.BlockSpec((D, D), lambda i: (0, 0))
    return pl.pallas_call(
        _tc1_body,
        grid=(N // _BLK,),
        in_specs=[
            pl.BlockSpec((_BLK, D), lambda i: (i, 0)),
            full, full, full, full, full, full,
            pl.BlockSpec((1, D), lambda i: (0, 0)),
        ],
        out_specs=[
            pl.BlockSpec((_BLK, D), lambda i: (i, 0)),
            pl.BlockSpec((_BLK, D), lambda i: (i, 0)),
            pl.BlockSpec((_BLK, D), lambda i: (i, 0)),
        ],
        out_shape=[
            jax.ShapeDtypeStruct((N, D), jnp.float32),
            jax.ShapeDtypeStruct((N, D), jnp.float32),
            jax.ShapeDtypeStruct((N, D), jnp.float32),
        ],
    )(x, wla_t, w1_t, wlb_t, w2_t, wra_t, wrb_t, bprime)


_MESH = plsc.VectorSubcoreMesh(core_axis_name="c", subcore_axis_name="s")


@functools.partial(
    pl.kernel,
    out_type=jax.ShapeDtypeStruct((NC, N_PAD, D), jnp.float32),
    mesh=_MESH,
    scratch_types=(
        [pltpu.VMEM_SHARED((N_PAD, D), jnp.float32)]
        + [pltpu.VMEM((SEG, CH), jnp.int32)] * 2
        + [pltpu.VMEM((QUAD, CH), jnp.int32)] * 2
        + [pltpu.VMEM((CH, D), jnp.float32)] * (2 * QUAD)
        + [pltpu.SemaphoreType.DMA] * (4 * QUAD)
    ),
)
def _sc_scatter(ya_hbm, yb_hbm, src_hbm, dst_hbm, zeros_hbm, out_hbm,
                acc, src_v, dst_v, mdst_v, msrc_v, *bufs):
    ra = bufs[0:QUAD]
    rb = bufs[QUAD:2 * QUAD]
    ga = bufs[2 * QUAD:3 * QUAD]
    gb = bufs[3 * QUAD:4 * QUAD]
    sa = bufs[4 * QUAD:5 * QUAD]
    sb = bufs[5 * QUAD:6 * QUAD]

    c = lax.axis_index("c")
    s = lax.axis_index("s")
    wid = s * NC + c

    pltpu.sync_copy(zeros_hbm.at[pl.ds(s * RPT, RPT)],
                    acc.at[pl.ds(s * RPT, RPT)])
    plsc.subcore_barrier()

    dummy = N + s

    def mask_chunk(jj, q):
        for k in range(CH // 16):
            sl = pl.ds(k * 16, 16)
            sv = src_v[jj, sl]
            dv = dst_v[jj, sl]
            m = sv == dv
            mdst_v[q, sl] = jnp.where(m, dummy, dv)
            msrc_v[q, sl] = jnp.where(m, dummy, sv)

    dummy_vec = jnp.zeros((16,), jnp.int32) + dummy
    for q in range(QUAD):
        for k in range(CH // 16):
            sl = pl.ds(k * 16, 16)
            mdst_v[q, sl] = dummy_vec
            msrc_v[q, sl] = dummy_vec
    for q in range(QUAD):
        pltpu.async_copy(ra[q], acc.at[mdst_v.at[q]], sa[q], add=True)
        pltpu.async_copy(rb[q], acc.at[msrc_v.at[q]], sb[q], add=True)

    def scat_wait(rows, ring, sem):
        pltpu.make_async_copy(rows, acc.at[ring], sem).wait()

    def quad(t, carry):
        j = QUAD * t
        da = []
        db = []
        for q in range(QUAD):
            scat_wait(ra[q], mdst_v.at[q], sa[q])
            da.append(pltpu.async_copy(ya_hbm.at[src_v.at[j + q]], ra[q], ga[q]))
            scat_wait(rb[q], msrc_v.at[q], sb[q])
            db.append(pltpu.async_copy(yb_hbm.at[dst_v.at[j + q]], rb[q], gb[q]))
            mask_chunk(j + q, q)
        for q in range(QUAD):
            da[q].wait()
            pltpu.async_copy(ra[q], acc.at[mdst_v.at[q]], sa[q], add=True)
            db[q].wait()
            pltpu.async_copy(rb[q], acc.at[msrc_v.at[q]], sb[q], add=True)
        return carry

    def segment(g, carry):
        pltpu.sync_copy(src_hbm.at[wid, pl.ds(g * SEG, SEG)], src_v)
        pltpu.sync_copy(dst_hbm.at[wid, pl.ds(g * SEG, SEG)], dst_v)
        return lax.fori_loop(0, BODIES, quad, carry)

    lax.fori_loop(0, NSEG, segment, 0)
    for q in range(QUAD):
        scat_wait(ra[q], mdst_v.at[q], sa[q])
        scat_wait(rb[q], msrc_v.at[q], sb[q])
    plsc.subcore_barrier()
    pltpu.sync_copy(acc.at[pl.ds(s * RPT, RPT)],
                    out_hbm.at[c, pl.ds(s * RPT, RPT)])


def _tc2_body(base_ref, a0_ref, a1_ref, out_ref):
    out_ref[...] = base_ref[...] + a0_ref[0] + a1_ref[0]


def _tc2(base, accs):
    spec = pl.BlockSpec((_BLK, D), lambda i: (i, 0))
    return pl.pallas_call(
        _tc2_body,
        grid=(N // _BLK,),
        in_specs=[
            spec,
            pl.BlockSpec((1, _BLK, D), lambda i: (0, i, 0)),
            pl.BlockSpec((1, _BLK, D), lambda i: (1, i, 0)),
        ],
        out_specs=spec,
        out_shape=jax.ShapeDtypeStruct((N, D), jnp.float32),
    )(base, accs, accs)


@jax.jit
def kernel(x, edge_index, Wl_a, bl_a, Wr_a, Wl_b, bl_b, Wr_b, W, b):
    w1 = W[:, :D]
    w2 = W[:, D:]
    bprime = (b + w1 @ bl_a + w2 @ bl_b)[None, :]

    ya, yb, base = _tc1(x, Wl_a.T, w1.T, Wl_b.T, w2.T, Wr_a.T, Wr_b.T, bprime)

    pad = jnp.broadcast_to(jnp.arange(EPW - E // NW, dtype=jnp.int32) * 97 % N,
                           (NW, EPW - E // NW))
    src = jnp.concatenate([edge_index[0].reshape(NW, E // NW), pad], axis=1)
    dst = jnp.concatenate([edge_index[1].reshape(NW, E // NW), pad], axis=1)
    src3 = src.reshape(NW, NCHUNK, CH)
    dst3 = dst.reshape(NW, NCHUNK, CH)
    zeros = jnp.zeros((N_PAD, D), jnp.float32)

    accs = _sc_scatter(ya, yb, src3, dst3, zeros)

    return _tc2(base, accs)
